# baseline (speedup 1.0000x reference)
import functools

import jax
import jax.numpy as jnp
from jax import lax
from jax.experimental import pallas as pl
from jax.experimental.pallas import tpu as pltpu
from jax.experimental.pallas import tpu_sc as plsc

_D = 64
_NC = 2
_NS = 16
_NW = _NC * _NS
_CHUNK = 128
_NBUF = 4


def _emb_lookup(x_flat, weights, b_per_w, nchunk):
    total = _NW * b_per_w
    nouter = nchunk // _NBUF
    mesh = plsc.VectorSubcoreMesh(core_axis_name="c", subcore_axis_name="s")

    @functools.partial(
        pl.kernel,
        out_type=jax.ShapeDtypeStruct((total, _D), jnp.float32),
        mesh=mesh,
        scratch_types=(
            [pltpu.VMEM((nchunk, _CHUNK), jnp.int32)]
            + [pltpu.VMEM((_NBUF, _CHUNK, 128), jnp.float32)]
            + [pltpu.SemaphoreType.DMA] * (2 * _NBUF)
        ),
        compiler_params=pltpu.CompilerParams(use_tc_tiling_on_sc=False),
    )
    def emb(x_hbm, tbl_hbm, out_hbm, idx_v, rows_v, *sems):
        gsem = sems[:_NBUF]
        wsem = sems[_NBUF:]
        wid = lax.axis_index("s") * _NC + lax.axis_index("c")
        base = wid * b_per_w
        pltpu.sync_copy(x_hbm.at[wid], idx_v)

        def gather_descr(c, b):
            return pltpu.make_async_copy(
                tbl_hbm.at[idx_v.at[c]], rows_v.at[b], gsem[b]
            )

        def write_descr(c, b):
            return pltpu.make_async_copy(
                rows_v.at[b, :, pl.ds(0, _D)],
                out_hbm.at[pl.ds(base + c * _CHUNK, _CHUNK)],
                wsem[b],
            )

        for b in range(_NBUF):
            gather_descr(b, b).start()

        def outer(o, carry):
            c0 = o * _NBUF
            for b in range(_NBUF):
                gather_descr(c0 + b, b).wait()
                write_descr(c0 + b, b).start()
            for b in range(_NBUF):
                write_descr(c0 + b, b).wait()
                gather_descr(c0 + b + _NBUF, b).start()
            return carry

        lax.fori_loop(0, nouter - 1, outer, 0)

        c0 = (nouter - 1) * _NBUF
        for b in range(_NBUF):
            gather_descr(c0 + b, b).wait()
            write_descr(c0 + b, b).start()
        for b in range(_NBUF):
            write_descr(c0 + b, b).wait()

    return emb(x_flat, weights)


def _t_w_body(inb, outb):
    outb[:, 0:_D] = inb[...].T


def _t_w(wt, n_rows):
    bp = 16384
    grid = (n_rows + bp - 1) // bp
    return pl.pallas_call(
        _t_w_body,
        grid=(grid,),
        in_specs=[pl.BlockSpec((_D, bp), lambda p: (0, p))],
        out_specs=pl.BlockSpec((bp, 128), lambda p: (p, 0)),
        out_shape=jax.ShapeDtypeStruct((n_rows, 128), jnp.float32),
    )(wt)


def _t_out_body(inb, outb):
    outb[...] = inb[...].T


def _t_out(m2, n_i, n_jc):
    bi, bj = 1024, 3200
    return pl.pallas_call(
        _t_out_body,
        grid=(n_i // bi, n_jc // bj),
        in_specs=[pl.BlockSpec((bi, bj), lambda i, j: (i, j))],
        out_specs=pl.BlockSpec((bj, bi), lambda i, j: (j, i)),
        out_shape=jax.ShapeDtypeStruct((n_jc, n_i), jnp.float32),
    )(m2)


def kernel(x, weights):
    batch = x.size
    n_i, n_j = x.shape
    n_rows = weights.shape[0]
    b_per_w = batch // _NW
    nchunk = b_per_w // _CHUNK
    x_flat = x.reshape(_NW, nchunk, _CHUNK)
    table = _t_w(weights.T, n_rows)
    rows = _emb_lookup(x_flat, table, b_per_w, nchunk)
    out_t = _t_out(rows.reshape(n_i, n_j * _D), n_i, n_j * _D)
    return jnp.transpose(out_t.reshape(n_j, _D, n_i), (2, 0, 1))


# speedup vs baseline: 1.0375x; 1.0375x over previous
import functools

import jax
import jax.numpy as jnp
from jax import lax
from jax.experimental import pallas as pl
from jax.experimental.pallas import tpu as pltpu
from jax.experimental.pallas import tpu_sc as plsc

_D = 64
_NC = 2
_NS = 16
_NW = _NC * _NS
_CHUNK = 64
_NBUF = 8


def _emb_lookup(x_flat, weights, b_per_w, nchunk):
    total = _NW * b_per_w
    nouter = nchunk // _NBUF
    mesh = plsc.VectorSubcoreMesh(core_axis_name="c", subcore_axis_name="s")

    @functools.partial(
        pl.kernel,
        out_type=jax.ShapeDtypeStruct((total, _D), jnp.float32),
        mesh=mesh,
        scratch_types=(
            [pltpu.VMEM((nchunk, _CHUNK), jnp.int32)]
            + [pltpu.VMEM((_NBUF, _CHUNK, 128), jnp.float32)]
            + [pltpu.SemaphoreType.DMA] * (2 * _NBUF)
        ),
        compiler_params=pltpu.CompilerParams(use_tc_tiling_on_sc=False),
    )
    def emb(x_hbm, tbl_hbm, out_hbm, idx_v, rows_v, *sems):
        gsem = sems[:_NBUF]
        wsem = sems[_NBUF:]
        wid = lax.axis_index("s") * _NC + lax.axis_index("c")
        base = wid * b_per_w
        pltpu.sync_copy(x_hbm.at[wid], idx_v)

        def gather_descr(c, b):
            return pltpu.make_async_copy(
                tbl_hbm.at[idx_v.at[c]], rows_v.at[b], gsem[b]
            )

        def write_descr(c, b):
            return pltpu.make_async_copy(
                rows_v.at[b, :, pl.ds(0, _D)],
                out_hbm.at[pl.ds(base + c * _CHUNK, _CHUNK)],
                wsem[b],
            )

        for b in range(_NBUF):
            gather_descr(b, b).start()

        def outer(o, carry):
            c0 = o * _NBUF
            for b in range(_NBUF):
                gather_descr(c0 + b, b).wait()
                write_descr(c0 + b, b).start()
            for b in range(_NBUF):
                write_descr(c0 + b, b).wait()
                gather_descr(c0 + b + _NBUF, b).start()
            return carry

        lax.fori_loop(0, nouter - 1, outer, 0)

        c0 = (nouter - 1) * _NBUF
        for b in range(_NBUF):
            gather_descr(c0 + b, b).wait()
            write_descr(c0 + b, b).start()
        for b in range(_NBUF):
            write_descr(c0 + b, b).wait()

    return emb(x_flat, weights)


def _t_w_body(inb, outb):
    outb[:, 0:_D] = inb[...].T


def _t_w(wt, n_rows):
    bp = 16384
    grid = (n_rows + bp - 1) // bp
    return pl.pallas_call(
        _t_w_body,
        grid=(grid,),
        in_specs=[pl.BlockSpec((_D, bp), lambda p: (0, p))],
        out_specs=pl.BlockSpec((bp, 128), lambda p: (p, 0)),
        out_shape=jax.ShapeDtypeStruct((n_rows, 128), jnp.float32),
    )(wt)


def _t_out_body(inb, outb):
    outb[...] = inb[...].T


def _t_out(m2, n_i, n_jc):
    bi, bj = 1024, 3200
    return pl.pallas_call(
        _t_out_body,
        grid=(n_i // bi, n_jc // bj),
        in_specs=[pl.BlockSpec((bi, bj), lambda i, j: (i, j))],
        out_specs=pl.BlockSpec((bj, bi), lambda i, j: (j, i)),
        out_shape=jax.ShapeDtypeStruct((n_jc, n_i), jnp.float32),
    )(m2)


def kernel(x, weights):
    batch = x.size
    n_i, n_j = x.shape
    n_rows = weights.shape[0]
    b_per_w = batch // _NW
    nchunk = b_per_w // _CHUNK
    x_flat = x.reshape(_NW, nchunk, _CHUNK)
    table = _t_w(weights.T, n_rows)
    rows = _emb_lookup(x_flat, table, b_per_w, nchunk)
    out_t = _t_out(rows.reshape(n_i, n_j * _D), n_i, n_j * _D)
    return jnp.transpose(out_t.reshape(n_j, _D, n_i), (2, 0, 1))


# speedup vs baseline: 1.0743x; 1.0355x over previous
import functools

import jax
import jax.numpy as jnp
from jax import lax
from jax.experimental import pallas as pl
from jax.experimental.pallas import tpu as pltpu
from jax.experimental.pallas import tpu_sc as plsc

_D = 64
_NC = 2
_NS = 16
_NW = _NC * _NS
_CHUNK = 64
_NBUF = 10


def _emb_lookup(x_flat, weights, b_per_w, nchunk):
    total = _NW * b_per_w
    nouter = nchunk // _NBUF
    mesh = plsc.VectorSubcoreMesh(core_axis_name="c", subcore_axis_name="s")

    @functools.partial(
        pl.kernel,
        out_type=jax.ShapeDtypeStruct((total, _D), jnp.float32),
        mesh=mesh,
        scratch_types=(
            [pltpu.VMEM((nchunk, _CHUNK), jnp.int32)]
            + [pltpu.VMEM((_NBUF, _CHUNK, 128), jnp.float32)]
            + [pltpu.SemaphoreType.DMA] * (2 * _NBUF)
        ),
        compiler_params=pltpu.CompilerParams(use_tc_tiling_on_sc=False),
    )
    def emb(x_hbm, tbl_hbm, out_hbm, idx_v, rows_v, *sems):
        gsem = sems[:_NBUF]
        wsem = sems[_NBUF:]
        wid = lax.axis_index("s") * _NC + lax.axis_index("c")
        base = wid * b_per_w
        pltpu.sync_copy(x_hbm.at[wid], idx_v)

        def gather_descr(c, b):
            return pltpu.make_async_copy(
                tbl_hbm.at[idx_v.at[c]], rows_v.at[b], gsem[b]
            )

        def write_descr(c, b):
            return pltpu.make_async_copy(
                rows_v.at[b, :, pl.ds(0, _D)],
                out_hbm.at[pl.ds(base + c * _CHUNK, _CHUNK)],
                wsem[b],
            )

        for b in range(_NBUF):
            gather_descr(b, b).start()

        def outer(o, carry):
            c0 = o * _NBUF
            for b in range(_NBUF):
                gather_descr(c0 + b, b).wait()
                write_descr(c0 + b, b).start()
            for b in range(_NBUF):
                write_descr(c0 + b, b).wait()
                gather_descr(c0 + b + _NBUF, b).start()
            return carry

        lax.fori_loop(0, nouter - 1, outer, 0)

        c0 = (nouter - 1) * _NBUF
        for b in range(_NBUF):
            gather_descr(c0 + b, b).wait()
            write_descr(c0 + b, b).start()
        for b in range(_NBUF):
            write_descr(c0 + b, b).wait()

    return emb(x_flat, weights)


def _t_w_body(inb, outb):
    outb[:, 0:_D] = inb[...].T


def _t_w(wt, n_rows):
    bp = 16384
    grid = (n_rows + bp - 1) // bp
    return pl.pallas_call(
        _t_w_body,
        grid=(grid,),
        in_specs=[pl.BlockSpec((_D, bp), lambda p: (0, p))],
        out_specs=pl.BlockSpec((bp, 128), lambda p: (p, 0)),
        out_shape=jax.ShapeDtypeStruct((n_rows, 128), jnp.float32),
    )(wt)


def _t_out_body(inb, outb):
    outb[...] = inb[...].T


def _t_out(m2, n_i, n_jc):
    bi, bj = 1024, 3200
    return pl.pallas_call(
        _t_out_body,
        grid=(n_i // bi, n_jc // bj),
        in_specs=[pl.BlockSpec((bi, bj), lambda i, j: (i, j))],
        out_specs=pl.BlockSpec((bj, bi), lambda i, j: (j, i)),
        out_shape=jax.ShapeDtypeStruct((n_jc, n_i), jnp.float32),
    )(m2)


def kernel(x, weights):
    batch = x.size
    n_i, n_j = x.shape
    n_rows = weights.shape[0]
    b_per_w = batch // _NW
    nchunk = b_per_w // _CHUNK
    x_flat = x.reshape(_NW, nchunk, _CHUNK)
    table = _t_w(weights.T, n_rows)
    rows = _emb_lookup(x_flat, table, b_per_w, nchunk)
    out_t = _t_out(rows.reshape(n_i, n_j * _D), n_i, n_j * _D)
    return jnp.transpose(out_t.reshape(n_j, _D, n_i), (2, 0, 1))


# speedup vs baseline: 1.0798x; 1.0051x over previous
import functools

import jax
import jax.numpy as jnp
from jax import lax
from jax.experimental import pallas as pl
from jax.experimental.pallas import tpu as pltpu
from jax.experimental.pallas import tpu_sc as plsc

_D = 64
_NC = 2
_NS = 16
_NW = _NC * _NS
_CHUNK = 64
_NBUF = 10


def _emb_lookup(x_flat, weights, b_per_w, nchunk):
    total = _NW * b_per_w
    nouter = nchunk // _NBUF
    mesh = plsc.VectorSubcoreMesh(core_axis_name="c", subcore_axis_name="s")

    @functools.partial(
        pl.kernel,
        out_type=jax.ShapeDtypeStruct((total, _D), jnp.float32),
        mesh=mesh,
        scratch_types=(
            [pltpu.VMEM((nchunk, _CHUNK), jnp.int32)]
            + [pltpu.VMEM((_NBUF, _CHUNK, 128), jnp.float32)]
            + [pltpu.SemaphoreType.DMA] * (2 * _NBUF)
        ),
        compiler_params=pltpu.CompilerParams(use_tc_tiling_on_sc=False),
    )
    def emb(x_hbm, tbl_hbm, out_hbm, idx_v, rows_v, *sems):
        gsem = sems[:_NBUF]
        wsem = sems[_NBUF:]
        wid = lax.axis_index("s") * _NC + lax.axis_index("c")
        base = wid * b_per_w
        pltpu.sync_copy(x_hbm.at[wid], idx_v)

        def gather_descr(c, b):
            return pltpu.make_async_copy(
                tbl_hbm.at[idx_v.at[c]], rows_v.at[b], gsem[b]
            )

        def write_descr(c, b):
            return pltpu.make_async_copy(
                rows_v.at[b, :, pl.ds(0, _D)],
                out_hbm.at[pl.ds(base + c * _CHUNK, _CHUNK)],
                wsem[b],
            )

        for b in range(_NBUF):
            gather_descr(b, b).start()

        def outer(o, carry):
            c0 = o * _NBUF
            for b in range(_NBUF):
                gather_descr(c0 + b, b).wait()
                write_descr(c0 + b, b).start()
            for b in range(_NBUF):
                write_descr(c0 + b, b).wait()
                gather_descr(c0 + b + _NBUF, b).start()
            return carry

        lax.fori_loop(0, nouter - 1, outer, 0)

        c0 = (nouter - 1) * _NBUF
        for b in range(_NBUF):
            gather_descr(c0 + b, b).wait()
            write_descr(c0 + b, b).start()
        for b in range(_NBUF):
            write_descr(c0 + b, b).wait()

    return emb(x_flat, weights)


def _t_w_body(inb, outb):
    outb[:, 0:_D] = inb[...].T


def _t_w(wt, n_rows):
    bp = 32768
    grid = (n_rows + bp - 1) // bp
    return pl.pallas_call(
        _t_w_body,
        grid=(grid,),
        in_specs=[pl.BlockSpec((_D, bp), lambda p: (0, p))],
        out_specs=pl.BlockSpec((bp, 128), lambda p: (p, 0)),
        out_shape=jax.ShapeDtypeStruct((n_rows, 128), jnp.float32),
    )(wt)


def _t_out_body(inb, outb):
    outb[...] = inb[...].T


def _t_out(m2, n_i, n_jc):
    bi, bj = 1024, 3200
    return pl.pallas_call(
        _t_out_body,
        grid=(n_i // bi, n_jc // bj),
        in_specs=[pl.BlockSpec((bi, bj), lambda i, j: (i, j))],
        out_specs=pl.BlockSpec((bj, bi), lambda i, j: (j, i)),
        out_shape=jax.ShapeDtypeStruct((n_jc, n_i), jnp.float32),
    )(m2)


def kernel(x, weights):
    batch = x.size
    n_i, n_j = x.shape
    n_rows = weights.shape[0]
    b_per_w = batch // _NW
    nchunk = b_per_w // _CHUNK
    x_flat = x.reshape(_NW, nchunk, _CHUNK)
    table = _t_w(weights.T, n_rows)
    rows = _emb_lookup(x_flat, table, b_per_w, nchunk)
    out_t = _t_out(rows.reshape(n_i, n_j * _D), n_i, n_j * _D)
    return jnp.transpose(out_t.reshape(n_j, _D, n_i), (2, 0, 1))
